# baseline (speedup 1.0000x reference)
import functools

import jax
import jax.numpy as jnp
from jax import lax
from jax.experimental import pallas as pl
from jax.experimental.pallas import tpu as pltpu
from jax.experimental.pallas import tpu_sc as plsc

N = 10000
E = 320000
D = 128

NC = 2
NS = 16
NW = NC * NS
K = 64
NP = 10112
EP = 327680
NCH = EP // (NW * K)
RPT = NP // NS


CW = 8


NBUF = 2


def _sc_body(x_hbm, src_hbm, dst_hbm, sums_hbm,
             src_v, dst_v, rows0, rows1,
             acc_sh, gs0, gs1, ss0, ss1):
    cid = lax.axis_index("c")
    sid = lax.axis_index("s")
    wid = cid * NS + sid
    rows = (rows0, rows1)
    gsem = (gs0, gs1)
    ssem = (ss0, ss1)

    pltpu.sync_copy(src_hbm.at[wid], src_v)
    pltpu.sync_copy(dst_hbm.at[wid], dst_v)

    zeros16 = jnp.zeros((16,), jnp.float32)

    def fill_rows(i, c):
        def inner(j, c2):
            rows0[i, pl.ds(j * 16, 16)] = zeros16
            return c2
        return lax.fori_loop(0, D // 16, inner, c)

    lax.fori_loop(0, K, fill_rows, 0)

    base = sid * RPT

    def zero_chunk(t, c):
        pltpu.sync_copy(rows0, acc_sh.at[pl.ds(base + t * K, K)])
        return c

    lax.fori_loop(0, RPT // K, zero_chunk, 0)
    if RPT % K:
        pltpu.sync_copy(rows0.at[pl.ds(0, RPT % K)],
                        acc_sh.at[pl.ds(base + (RPT // K) * K, RPT % K)])
    plsc.subcore_barrier()

    def gfire(b, j):
        pltpu.async_copy(x_hbm.at[src_v.at[j]], rows[b], gsem[b])

    def gwait(b, j):
        pltpu.make_async_copy(x_hbm.at[src_v.at[j]], rows[b], gsem[b]).wait()

    def sfire(b, j):
        pltpu.async_copy(rows[b], acc_sh.at[dst_v.at[j]], ssem[b], add=True)

    def swait(b, j):
        pltpu.make_async_copy(rows[b], acc_sh.at[dst_v.at[j]],
                              ssem[b]).wait()

    gfire(0, 0)

    def group(g, c):
        for b in range(NBUF):
            j = g * NBUF + b

            @pl.when(j < NCH)
            def _():
                gwait(b, j)
                sfire(b, j)

            @pl.when(jnp.logical_and(j >= 1, j <= NCH))
            def _():
                swait(1 - b, j - 1)

            @pl.when(j + 1 < NCH)
            def _():
                gfire(1 - b, j + 1)
        return c

    lax.fori_loop(0, (NCH + 2) // NBUF, group, 0)

    plsc.subcore_barrier()

    pltpu.sync_copy(acc_sh.at[pl.ds(base, RPT)],
                    sums_hbm.at[cid, pl.ds(base, RPT)])


_sc_aggregate = functools.partial(
    pl.kernel,
    out_type=jax.ShapeDtypeStruct((NC, NP, D), jnp.float32),
    mesh=plsc.VectorSubcoreMesh(core_axis_name="c", subcore_axis_name="s"),
    compiler_params=pltpu.CompilerParams(use_tc_tiling_on_sc=False),
    scratch_types=(
        pltpu.VMEM((NCH, K), jnp.int32),
        pltpu.VMEM((NCH, K), jnp.int32),
        pltpu.VMEM((K, D), jnp.float32),
        pltpu.VMEM((K, D), jnp.float32),
        pltpu.VMEM_SHARED((NP, D), jnp.float32),
        pltpu.SemaphoreType.DMA,
        pltpu.SemaphoreType.DMA,
        pltpu.SemaphoreType.DMA,
        pltpu.SemaphoreType.DMA,
    ),
)(_sc_body)


def _sc_count_body(dst_hbm, ones_hbm, zc_hbm, cnts_hbm,
                   dst_v, ones_v, cnt_sh):
    cid = lax.axis_index("c")
    sid = lax.axis_index("s")
    wid = cid * NS + sid

    pltpu.sync_copy(dst_hbm.at[wid], dst_v)
    pltpu.sync_copy(ones_hbm, ones_v)
    base = sid * RPT
    pltpu.sync_copy(zc_hbm, cnt_sh.at[pl.ds(base, RPT)])
    plsc.subcore_barrier()

    def chunk(j, c):
        pltpu.sync_copy(ones_v, cnt_sh.at[dst_v.at[j]], add=True)
        return c

    lax.fori_loop(0, NCH, chunk, 0)

    plsc.subcore_barrier()
    pltpu.sync_copy(cnt_sh.at[pl.ds(base, RPT)],
                    cnts_hbm.at[cid, pl.ds(base, RPT)])


_sc_count = functools.partial(
    pl.kernel,
    out_type=jax.ShapeDtypeStruct((NC, NP, CW), jnp.float32),
    mesh=plsc.VectorSubcoreMesh(core_axis_name="c", subcore_axis_name="s"),
    compiler_params=pltpu.CompilerParams(use_tc_tiling_on_sc=False),
    scratch_types=(
        pltpu.VMEM((NCH, K), jnp.int32),
        pltpu.VMEM((K, CW), jnp.float32),
        pltpu.VMEM_SHARED((NP, CW), jnp.float32),
    ),
)(_sc_count_body)


def _tc_body(x_ref, p0_ref, p1_ref, c0_ref, c1_ref, wl_ref, wr_ref, b_ref,
             o_ref):
    x = x_ref[...]
    s = p0_ref[...] + p1_ref[...]
    c = jnp.maximum(c0_ref[...][:, 0:1] + c1_ref[...][:, 0:1], 1.0)
    agg = s / c
    y = (jnp.dot(agg, wl_ref[...], preferred_element_type=jnp.float32)
         + b_ref[...]
         + jnp.dot(x, wr_ref[...], preferred_element_type=jnp.float32))
    o_ref[...] = x + jnp.maximum(y, 0.0)


BT = 1000


def _tc_combine(x, p0, p1, c0, c1, wlT, wrT, b):
    grid = (N // BT,)
    return pl.pallas_call(
        _tc_body,
        out_shape=jax.ShapeDtypeStruct((N, D), jnp.float32),
        grid=grid,
        in_specs=[
            pl.BlockSpec((BT, D), lambda i: (i, 0)),
            pl.BlockSpec((BT, D), lambda i: (i, 0)),
            pl.BlockSpec((BT, D), lambda i: (i, 0)),
            pl.BlockSpec((BT, CW), lambda i: (i, 0)),
            pl.BlockSpec((BT, CW), lambda i: (i, 0)),
            pl.BlockSpec((D, D), lambda i: (0, 0)),
            pl.BlockSpec((D, D), lambda i: (0, 0)),
            pl.BlockSpec((1, D), lambda i: (0, 0)),
        ],
        out_specs=pl.BlockSpec((BT, D), lambda i: (i, 0)),
    )(x, p0, p1, c0, c1, wlT, wrT, b)


def kernel(x, edge_index, W_l, b_l, W_r):
    pad = EP - E
    src = jnp.concatenate(
        [edge_index[0], jnp.zeros((pad,), jnp.int32)]).reshape(NW, NCH, K)
    dst = jnp.concatenate(
        [edge_index[1], jnp.full((pad,), N, jnp.int32)]).reshape(NW, NCH, K)
    ones_col = jnp.ones((K, CW), jnp.float32)
    zeros_col = jnp.zeros((RPT, CW), jnp.float32)
    sums = _sc_aggregate(x, src, dst)
    cnts = _sc_count(dst, ones_col, zeros_col)
    return _tc_combine(x, sums[0], sums[1], cnts[0], cnts[1],
                       W_l.T, W_r.T, b_l.reshape(1, D))


# speedup vs baseline: 2.6016x; 2.6016x over previous
import functools

import jax
import jax.numpy as jnp
from jax import lax
from jax.experimental import pallas as pl
from jax.experimental.pallas import tpu as pltpu
from jax.experimental.pallas import tpu_sc as plsc

N = 10000
E = 320000
D = 128
HD = D // 2

NC = 2
NS = 16
NW = NC * NS
K = 64
NP = 10112
EP = 327680
NCH = EP // (NS * K)
NCHC = EP // (NW * K)
RPT = NP // NS
XRT = N // NS

CW = 8
NBUF = 2


def _sc_body(xs_hbm, src_hbm, dst_hbm, sums_hbm,
             src_v, dst_v, rows0, rows1, x_sh, acc_sh, gs0, gs1):
    cid = lax.axis_index("c")
    sid = lax.axis_index("s")
    rows = (rows0, rows1)
    gsem = (gs0, gs1)

    pltpu.sync_copy(src_hbm.at[sid], src_v)
    pltpu.sync_copy(dst_hbm.at[sid], dst_v)

    pltpu.sync_copy(xs_hbm.at[cid, pl.ds(sid * XRT, XRT)],
                    x_sh.at[pl.ds(sid * XRT, XRT)])

    zeros16 = jnp.zeros((16,), jnp.float32)

    def fill_rows(i, c):
        def inner(j, c2):
            rows0[i, pl.ds(j * 16, 16)] = zeros16
            return c2
        return lax.fori_loop(0, HD // 16, inner, c)

    lax.fori_loop(0, K, fill_rows, 0)

    base = sid * RPT

    def zero_chunk(t, c):
        pltpu.sync_copy(rows0, acc_sh.at[pl.ds(base + t * K, K)])
        return c

    lax.fori_loop(0, RPT // K, zero_chunk, 0)
    if RPT % K:
        pltpu.sync_copy(rows0.at[pl.ds(0, RPT % K)],
                        acc_sh.at[pl.ds(base + (RPT // K) * K, RPT % K)])
    plsc.subcore_barrier()

    def gfire(b, j):
        pltpu.async_copy(x_sh.at[src_v.at[j]], rows[b], gsem[b])

    def gwait(b, j):
        pltpu.make_async_copy(x_sh.at[src_v.at[j]], rows[b], gsem[b]).wait()

    gfire(0, 0)

    def group(g, c):
        for b in range(NBUF):
            j = g * NBUF + b

            @pl.when(j + 1 < NCH)
            def _():
                gfire(1 - b, j + 1)

            gwait(b, j)
            pltpu.sync_copy(rows[b], acc_sh.at[dst_v.at[j]], add=True)
        return c

    lax.fori_loop(0, NCH // NBUF, group, 0)

    plsc.subcore_barrier()

    pltpu.sync_copy(acc_sh.at[pl.ds(base, RPT)],
                    sums_hbm.at[cid, pl.ds(base, RPT)])


_sc_aggregate = functools.partial(
    pl.kernel,
    out_type=jax.ShapeDtypeStruct((NC, NP, HD), jnp.float32),
    mesh=plsc.VectorSubcoreMesh(core_axis_name="c", subcore_axis_name="s"),
    compiler_params=pltpu.CompilerParams(use_tc_tiling_on_sc=False),
    scratch_types=(
        pltpu.VMEM((NCH, K), jnp.int32),
        pltpu.VMEM((NCH, K), jnp.int32),
        pltpu.VMEM((K, HD), jnp.float32),
        pltpu.VMEM((K, HD), jnp.float32),
        pltpu.VMEM_SHARED((N, HD), jnp.float32),
        pltpu.VMEM_SHARED((NP, HD), jnp.float32),
        pltpu.SemaphoreType.DMA,
        pltpu.SemaphoreType.DMA,
    ),
)(_sc_body)


def _sc_count_body(dst_hbm, ones_hbm, zc_hbm, cnts_hbm,
                   dst_v, ones_v, cnt_sh):
    cid = lax.axis_index("c")
    sid = lax.axis_index("s")
    wid = cid * NS + sid

    pltpu.sync_copy(dst_hbm.at[wid], dst_v)
    pltpu.sync_copy(ones_hbm, ones_v)
    base = sid * RPT
    pltpu.sync_copy(zc_hbm, cnt_sh.at[pl.ds(base, RPT)])
    plsc.subcore_barrier()

    def chunk(j, c):
        pltpu.sync_copy(ones_v, cnt_sh.at[dst_v.at[j]], add=True)
        return c

    lax.fori_loop(0, NCHC, chunk, 0)

    plsc.subcore_barrier()
    pltpu.sync_copy(cnt_sh.at[pl.ds(base, RPT)],
                    cnts_hbm.at[cid, pl.ds(base, RPT)])


_sc_count = functools.partial(
    pl.kernel,
    out_type=jax.ShapeDtypeStruct((NC, NP, CW), jnp.float32),
    mesh=plsc.VectorSubcoreMesh(core_axis_name="c", subcore_axis_name="s"),
    compiler_params=pltpu.CompilerParams(use_tc_tiling_on_sc=False),
    scratch_types=(
        pltpu.VMEM((NCHC, K), jnp.int32),
        pltpu.VMEM((K, CW), jnp.float32),
        pltpu.VMEM_SHARED((NP, CW), jnp.float32),
    ),
)(_sc_count_body)


def _tc_body(x_ref, p0_ref, p1_ref, c0_ref, c1_ref, wl_ref, wr_ref, b_ref,
             o_ref):
    x = x_ref[...]
    s = jnp.concatenate([p0_ref[...], p1_ref[...]], axis=1)
    c = jnp.maximum(c0_ref[...][:, 0:1] + c1_ref[...][:, 0:1], 1.0)
    agg = s / c
    y = (jnp.dot(agg, wl_ref[...], preferred_element_type=jnp.float32)
         + b_ref[...]
         + jnp.dot(x, wr_ref[...], preferred_element_type=jnp.float32))
    o_ref[...] = x + jnp.maximum(y, 0.0)


BT = 1000


def _tc_combine(x, p0, p1, c0, c1, wlT, wrT, b):
    grid = (N // BT,)
    return pl.pallas_call(
        _tc_body,
        out_shape=jax.ShapeDtypeStruct((N, D), jnp.float32),
        grid=grid,
        in_specs=[
            pl.BlockSpec((BT, D), lambda i: (i, 0)),
            pl.BlockSpec((BT, HD), lambda i: (i, 0)),
            pl.BlockSpec((BT, HD), lambda i: (i, 0)),
            pl.BlockSpec((BT, CW), lambda i: (i, 0)),
            pl.BlockSpec((BT, CW), lambda i: (i, 0)),
            pl.BlockSpec((D, D), lambda i: (0, 0)),
            pl.BlockSpec((D, D), lambda i: (0, 0)),
            pl.BlockSpec((1, D), lambda i: (0, 0)),
        ],
        out_specs=pl.BlockSpec((BT, D), lambda i: (i, 0)),
    )(x, p0, p1, c0, c1, wlT, wrT, b)


def kernel(x, edge_index, W_l, b_l, W_r):
    pad = EP - E
    src_pad = jnp.concatenate([edge_index[0], jnp.zeros((pad,), jnp.int32)])
    dst_pad = jnp.concatenate([edge_index[1], jnp.full((pad,), N, jnp.int32)])
    src_a = src_pad.reshape(NS, NCH, K)
    dst_a = dst_pad.reshape(NS, NCH, K)
    dst_c = dst_pad.reshape(NW, NCHC, K)
    xs = jnp.stack([x[:, :HD], x[:, HD:]])
    ones_col = jnp.ones((K, CW), jnp.float32)
    zeros_col = jnp.zeros((RPT, CW), jnp.float32)
    sums = _sc_aggregate(xs, src_a, dst_a)
    cnts = _sc_count(dst_c, ones_col, zeros_col)
    return _tc_combine(x, sums[0], sums[1], cnts[0], cnts[1],
                       W_l.T, W_r.T, b_l.reshape(1, D))


# speedup vs baseline: 2.7807x; 1.0688x over previous
import functools

import jax
import jax.numpy as jnp
from jax import lax
from jax.experimental import pallas as pl
from jax.experimental.pallas import tpu as pltpu
from jax.experimental.pallas import tpu_sc as plsc

N = 10000
E = 320000
D = 128
HD = D // 2

NC = 2
NS = 16
NW = NC * NS
K = 64
NP = 10112
EP = 327680
NCH = EP // (NS * K)
NCHC = EP // (NW * K)
RPT = NP // NS
XRT = N // NS

CW = 8
NBUF = 2


def _sc_body(xs_hbm, src_hbm, dst_hbm, sums_hbm,
             src_v, dst_v, rows0, rows1, x_sh, acc_sh, gs0, gs1):
    cid = lax.axis_index("c")
    sid = lax.axis_index("s")
    rows = (rows0, rows1)
    gsem = (gs0, gs1)

    pltpu.sync_copy(src_hbm.at[sid], src_v)
    pltpu.sync_copy(dst_hbm.at[sid], dst_v)

    pltpu.sync_copy(xs_hbm.at[pl.ds(sid * XRT, XRT), pl.ds(cid * HD, HD)],
                    x_sh.at[pl.ds(sid * XRT, XRT)])

    zeros16 = jnp.zeros((16,), jnp.float32)

    def fill_rows(i, c):
        def inner(j, c2):
            rows0[i, pl.ds(j * 16, 16)] = zeros16
            return c2
        return lax.fori_loop(0, HD // 16, inner, c)

    lax.fori_loop(0, K, fill_rows, 0)

    base = sid * RPT

    def zero_chunk(t, c):
        pltpu.sync_copy(rows0, acc_sh.at[pl.ds(base + t * K, K)])
        return c

    lax.fori_loop(0, RPT // K, zero_chunk, 0)
    if RPT % K:
        pltpu.sync_copy(rows0.at[pl.ds(0, RPT % K)],
                        acc_sh.at[pl.ds(base + (RPT // K) * K, RPT % K)])
    plsc.subcore_barrier()

    def gfire(b, j):
        pltpu.async_copy(x_sh.at[src_v.at[j]], rows[b], gsem[b])

    def gwait(b, j):
        pltpu.make_async_copy(x_sh.at[src_v.at[j]], rows[b], gsem[b]).wait()

    gfire(0, 0)

    def group(g, c):
        for b in range(NBUF):
            j = g * NBUF + b

            @pl.when(j + 1 < NCH)
            def _():
                gfire(1 - b, j + 1)

            gwait(b, j)
            pltpu.sync_copy(rows[b], acc_sh.at[dst_v.at[j]], add=True)
        return c

    lax.fori_loop(0, NCH // NBUF, group, 0)

    plsc.subcore_barrier()

    pltpu.sync_copy(acc_sh.at[pl.ds(base, RPT)],
                    sums_hbm.at[cid, pl.ds(base, RPT)])


_sc_aggregate = functools.partial(
    pl.kernel,
    out_type=jax.ShapeDtypeStruct((NC, NP, HD), jnp.float32),
    mesh=plsc.VectorSubcoreMesh(core_axis_name="c", subcore_axis_name="s"),
    compiler_params=pltpu.CompilerParams(use_tc_tiling_on_sc=False),
    scratch_types=(
        pltpu.VMEM((NCH, K), jnp.int32),
        pltpu.VMEM((NCH, K), jnp.int32),
        pltpu.VMEM((K, HD), jnp.float32),
        pltpu.VMEM((K, HD), jnp.float32),
        pltpu.VMEM_SHARED((N, HD), jnp.float32),
        pltpu.VMEM_SHARED((NP, HD), jnp.float32),
        pltpu.SemaphoreType.DMA,
        pltpu.SemaphoreType.DMA,
    ),
)(_sc_body)


def _sc_count_body(dst_hbm, ones_hbm, zc_hbm, cnts_hbm,
                   dst_v, ones_v, cnt_sh):
    cid = lax.axis_index("c")
    sid = lax.axis_index("s")
    wid = cid * NS + sid

    pltpu.sync_copy(dst_hbm.at[wid], dst_v)
    pltpu.sync_copy(ones_hbm, ones_v)
    base = sid * RPT
    pltpu.sync_copy(zc_hbm, cnt_sh.at[pl.ds(base, RPT)])
    plsc.subcore_barrier()

    def chunk(j, c):
        pltpu.sync_copy(ones_v, cnt_sh.at[dst_v.at[j]], add=True)
        return c

    lax.fori_loop(0, NCHC, chunk, 0)

    plsc.subcore_barrier()
    pltpu.sync_copy(cnt_sh.at[pl.ds(base, RPT)],
                    cnts_hbm.at[cid, pl.ds(base, RPT)])


_sc_count = functools.partial(
    pl.kernel,
    out_type=jax.ShapeDtypeStruct((NC, NP, CW), jnp.float32),
    mesh=plsc.VectorSubcoreMesh(core_axis_name="c", subcore_axis_name="s"),
    compiler_params=pltpu.CompilerParams(use_tc_tiling_on_sc=False),
    scratch_types=(
        pltpu.VMEM((NCHC, K), jnp.int32),
        pltpu.VMEM((K, CW), jnp.float32),
        pltpu.VMEM_SHARED((NP, CW), jnp.float32),
    ),
)(_sc_count_body)


def _tc_body(x_ref, p0_ref, p1_ref, c0_ref, c1_ref, wl_ref, wr_ref, b_ref,
             o_ref):
    x = x_ref[...]
    s = jnp.concatenate([p0_ref[...], p1_ref[...]], axis=1)
    c = jnp.maximum(c0_ref[...][:, 0:1] + c1_ref[...][:, 0:1], 1.0)
    agg = s / c
    y = (jnp.dot(agg, wl_ref[...], preferred_element_type=jnp.float32)
         + b_ref[...]
         + jnp.dot(x, wr_ref[...], preferred_element_type=jnp.float32))
    o_ref[...] = x + jnp.maximum(y, 0.0)


BT = 1000


def _tc_combine(x, p0, p1, c0, c1, wlT, wrT, b):
    grid = (N // BT,)
    return pl.pallas_call(
        _tc_body,
        out_shape=jax.ShapeDtypeStruct((N, D), jnp.float32),
        grid=grid,
        in_specs=[
            pl.BlockSpec((BT, D), lambda i: (i, 0)),
            pl.BlockSpec((BT, HD), lambda i: (i, 0)),
            pl.BlockSpec((BT, HD), lambda i: (i, 0)),
            pl.BlockSpec((BT, CW), lambda i: (i, 0)),
            pl.BlockSpec((BT, CW), lambda i: (i, 0)),
            pl.BlockSpec((D, D), lambda i: (0, 0)),
            pl.BlockSpec((D, D), lambda i: (0, 0)),
            pl.BlockSpec((1, D), lambda i: (0, 0)),
        ],
        out_specs=pl.BlockSpec((BT, D), lambda i: (i, 0)),
    )(x, p0, p1, c0, c1, wlT, wrT, b)


def kernel(x, edge_index, W_l, b_l, W_r):
    pad = EP - E
    src_pad = jnp.concatenate([edge_index[0], jnp.zeros((pad,), jnp.int32)])
    dst_pad = jnp.concatenate([edge_index[1], jnp.full((pad,), N, jnp.int32)])
    src_a = src_pad.reshape(NS, NCH, K)
    dst_a = dst_pad.reshape(NS, NCH, K)
    dst_c = dst_pad.reshape(NW, NCHC, K)
    ones_col = jnp.ones((K, CW), jnp.float32)
    zeros_col = jnp.zeros((RPT, CW), jnp.float32)
    sums = _sc_aggregate(x, src_a, dst_a)
    cnts = _sc_count(dst_c, ones_col, zeros_col)
    return _tc_combine(x, sums[0], sums[1], cnts[0], cnts[1],
                       W_l.T, W_r.T, b_l.reshape(1, D))
